# Initial kernel scaffold; baseline (speedup 1.0000x reference)
import jax
import jax.numpy as jnp
from jax.experimental import pallas as pl


def kernel(c, b, t, f, char_emb, bi_emb, tri_emb, Wp, bp, W1, b1, W2, b2):
    raise NotImplementedError("write your pallas kernel here")


# speedup vs baseline: 5.5104x; 5.5104x over previous
import jax
import jax.numpy as jnp
from jax import lax
from jax.experimental import pallas as pl
from jax.experimental.pallas import tpu as pltpu
from jax.experimental.pallas import tpu_sc as plsc

EMBED = 64
PADE = 128
SEQ = 200
BATCH = 16384
FEAT = 32

NC, NS = 2, 16
NW = NC * NS
RPT = BATCH // NW
IPT = RPT * SEQ
GW = 128
SPB = 8
BLK = SPB * GW
NBLK = IPT // BLK
IDX_ROWS_PT = IPT // GW


def _sc_pool_kernel(c2, b2, t2, tab_c, tab_b, tab_t):
    mesh = plsc.VectorSubcoreMesh(core_axis_name="c", subcore_axis_name="s",
                                  num_cores=NC, num_subcores=NS)

    def body(c_hbm, b_hbm, t_hbm, tabc, tabb, tabt,
             out_c, out_b, out_t,
             idx_v, rows0, rows1, seg_v, acc, sem0, sem1):
        cid = lax.axis_index("c")
        sid = lax.axis_index("s")
        gtile = cid * NS + sid
        accbase = sid * RPT
        gbase = gtile * RPT
        idx_row0 = gtile * IDX_ROWS_PT

        zero = jnp.zeros((16,), jnp.float32)

        def zbody(i, carry):
            for h in range(PADE // 16):
                rows0[i, pl.ds(h * 16, 16)] = zero
            return carry

        lax.fori_loop(0, GW, zbody, 0)

        for idx_hbm, tab, out in (
            (c_hbm, tabc, out_c),
            (b_hbm, tabb, out_b),
            (t_hbm, tabt, out_t),
        ):
            for k in range(RPT // GW):
                pltpu.sync_copy(rows0, acc.at[pl.ds(accbase + k * GW, GW)])

            def blk_body(blk, carry, idx_hbm=idx_hbm, tab=tab):
                pltpu.sync_copy(
                    idx_hbm.at[pl.ds(idx_row0 + blk * SPB, SPB)], idx_v)
                descs = [None] * SPB
                descs[0] = pltpu.async_copy(tab.at[idx_v.at[0]], rows0, sem0)
                for j in range(SPB):
                    if j + 1 < SPB:
                        nbuf = rows0 if (j + 1) % 2 == 0 else rows1
                        nsem = sem0 if (j + 1) % 2 == 0 else sem1
                        descs[j + 1] = pltpu.async_copy(
                            tab.at[idx_v.at[j + 1]], nbuf, nsem)
                    descs[j].wait()
                    p0 = blk * BLK + j * GW
                    for h in range(GW // 16):
                        v = p0 + h * 16 + lax.iota(jnp.int32, 16)
                        q = lax.div(v, jnp.full((16,), SEQ, jnp.int32))
                        seg_v[pl.ds(h * 16, 16)] = accbase + q
                    src = rows0 if j % 2 == 0 else rows1
                    pltpu.sync_copy(src, acc.at[seg_v], add=True)
                return carry

            lax.fori_loop(0, NBLK, blk_body, 0)
            pltpu.sync_copy(acc.at[pl.ds(accbase, RPT)],
                            out.at[pl.ds(gbase, RPT)])

    f32 = jnp.float32
    return pl.kernel(
        body,
        out_type=[jax.ShapeDtypeStruct((BATCH, PADE), f32)] * 3,
        mesh=mesh,
        scratch_types=[
            pltpu.VMEM((SPB, GW), jnp.int32),
            pltpu.VMEM((GW, PADE), f32),
            pltpu.VMEM((GW, PADE), f32),
            pltpu.VMEM((GW,), jnp.int32),
            pltpu.VMEM_SHARED((NS * RPT, PADE), f32),
            pltpu.SemaphoreType.DMA,
            pltpu.SemaphoreType.DMA,
        ],
    )(c2, b2, t2, tab_c, tab_b, tab_t)


BM = 512


def _mlp_body(sc_ref, sb_ref, st_ref, f_ref, wpT_ref, bp_ref,
              w1T_ref, b1_ref, w2T_ref, b2_ref, out_ref):
    f32 = jnp.float32
    fe = jnp.dot(f_ref[...], wpT_ref[...], preferred_element_type=f32)
    fe = fe + bp_ref[...]
    xm = jnp.concatenate([sc_ref[:, :EMBED], sb_ref[:, :EMBED],
                          st_ref[:, :EMBED]], axis=1)
    x = jnp.concatenate([xm * (1.0 / SEQ), fe], axis=1)
    h = jnp.dot(x, w1T_ref[...], preferred_element_type=f32) + b1_ref[...]
    h = jnp.maximum(h, 0.0)
    out_ref[...] = jnp.dot(h, w2T_ref[...],
                           preferred_element_type=f32) + b2_ref[...]


def _mlp(Sc, Sb, St, f, WpT, bp2, W1T, b12, W2T, b22):
    nb = BATCH // BM
    full = lambda r, c: pl.BlockSpec((r, c), lambda i: (0, 0))
    return pl.pallas_call(
        _mlp_body,
        grid=(nb,),
        in_specs=[
            pl.BlockSpec((BM, PADE), lambda i: (i, 0)),
            pl.BlockSpec((BM, PADE), lambda i: (i, 0)),
            pl.BlockSpec((BM, PADE), lambda i: (i, 0)),
            pl.BlockSpec((BM, FEAT), lambda i: (i, 0)),
            full(FEAT, EMBED),
            full(1, EMBED),
            full(4 * EMBED, 128),
            full(1, 128),
            full(128, 2),
            full(1, 2),
        ],
        out_specs=pl.BlockSpec((BM, 2), lambda i: (i, 0)),
        out_shape=jax.ShapeDtypeStruct((BATCH, 2), jnp.float32),
    )(Sc, Sb, St, f, WpT, bp2, W1T, b12, W2T, b22)


def _pad_table(tab):
    return jnp.pad(tab, ((0, 0), (0, PADE - tab.shape[1])))


def kernel(c, b, t, f, char_emb, bi_emb, tri_emb, Wp, bp, W1, b1, W2, b2):
    c2 = c.astype(jnp.int32).reshape(-1, GW)
    b2i = b.astype(jnp.int32).reshape(-1, GW)
    t2 = t.astype(jnp.int32).reshape(-1, GW)
    Sc, Sb, St = _sc_pool_kernel(c2, b2i, t2, _pad_table(char_emb),
                                 _pad_table(bi_emb), _pad_table(tri_emb))
    return _mlp(Sc, Sb, St, f,
                Wp.T, bp.reshape(1, -1),
                W1.T, b1.reshape(1, -1),
                W2.T, b2.reshape(1, -1))


# speedup vs baseline: 5.6102x; 1.0181x over previous
import jax
import jax.numpy as jnp
from jax import lax
from jax.experimental import pallas as pl
from jax.experimental.pallas import tpu as pltpu
from jax.experimental.pallas import tpu_sc as plsc

EMBED = 64
PADE = 128
SEQ = 200
BATCH = 16384
FEAT = 32

NC, NS = 2, 16
NW = NC * NS
RPT = BATCH // NW
IPT = RPT * SEQ
GW = 128
SPB = 8
BLK = SPB * GW
NBLK = IPT // BLK
IDX_ROWS_PT = IPT // GW


def _sc_pool_kernel(c2, b2, t2, tab_c, tab_b, tab_t):
    mesh = plsc.VectorSubcoreMesh(core_axis_name="c", subcore_axis_name="s",
                                  num_cores=NC, num_subcores=NS)

    def body(c_hbm, b_hbm, t_hbm, tabc, tabb, tabt,
             out_c, out_b, out_t,
             idx_v, rows0, rows1, zbuf, seg_v, acc,
             sem0, sem1, ssem0, ssem1):
        cid = lax.axis_index("c")
        sid = lax.axis_index("s")
        gtile = cid * NS + sid
        accbase = sid * RPT
        gbase = gtile * RPT
        idx_row0 = gtile * IDX_ROWS_PT

        zero = jnp.zeros((16,), jnp.float32)

        def zbody(i, carry):
            for h in range(PADE // 16):
                zbuf[i, pl.ds(h * 16, 16)] = zero
            return carry

        lax.fori_loop(0, GW, zbody, 0)
        for h in range(GW // 16):
            seg_v[pl.ds(h * 16, 16)] = accbase + lax.iota(jnp.int32, 16) * 0

        bufs = (rows0, rows1)
        gsems = (sem0, sem1)
        ssems = (ssem0, ssem1)

        def scat_wait(par):
            pltpu.make_async_copy(bufs[par], acc.at[seg_v], ssems[par]).wait()

        for idx_hbm, tab, out in (
            (c_hbm, tabc, out_c),
            (b_hbm, tabb, out_b),
            (t_hbm, tabt, out_t),
        ):
            for k in range(RPT // GW):
                pltpu.sync_copy(zbuf, acc.at[pl.ds(accbase + k * GW, GW)])
            pltpu.async_copy(zbuf, acc.at[seg_v], ssems[0], add=True)
            pltpu.async_copy(zbuf, acc.at[seg_v], ssems[1], add=True)

            def blk_body(blk, carry, idx_hbm=idx_hbm, tab=tab):
                pltpu.sync_copy(
                    idx_hbm.at[pl.ds(idx_row0 + blk * SPB, SPB)], idx_v)
                descs = [None] * SPB
                scat_wait(0)
                descs[0] = pltpu.async_copy(tab.at[idx_v.at[0]], rows0, sem0)
                for j in range(SPB):
                    if j + 1 < SPB:
                        scat_wait((j + 1) % 2)
                        descs[j + 1] = pltpu.async_copy(
                            tab.at[idx_v.at[j + 1]], bufs[(j + 1) % 2],
                            gsems[(j + 1) % 2])
                    descs[j].wait()
                    p0 = blk * BLK + j * GW
                    for h in range(GW // 16):
                        v = p0 + h * 16 + lax.iota(jnp.int32, 16)
                        q = lax.div(v, jnp.full((16,), SEQ, jnp.int32))
                        seg_v[pl.ds(h * 16, 16)] = accbase + q
                    pltpu.async_copy(bufs[j % 2], acc.at[seg_v],
                                     ssems[j % 2], add=True)
                return carry

            lax.fori_loop(0, NBLK, blk_body, 0)
            scat_wait(0)
            scat_wait(1)
            pltpu.sync_copy(acc.at[pl.ds(accbase, RPT)],
                            out.at[pl.ds(gbase, RPT)])

    f32 = jnp.float32
    return pl.kernel(
        body,
        out_type=[jax.ShapeDtypeStruct((BATCH, PADE), f32)] * 3,
        mesh=mesh,
        scratch_types=[
            pltpu.VMEM((SPB, GW), jnp.int32),
            pltpu.VMEM((GW, PADE), f32),
            pltpu.VMEM((GW, PADE), f32),
            pltpu.VMEM((GW, PADE), f32),
            pltpu.VMEM((GW,), jnp.int32),
            pltpu.VMEM_SHARED((NS * RPT, PADE), f32),
            pltpu.SemaphoreType.DMA,
            pltpu.SemaphoreType.DMA,
            pltpu.SemaphoreType.DMA,
            pltpu.SemaphoreType.DMA,
        ],
    )(c2, b2, t2, tab_c, tab_b, tab_t)


BM = 512


def _mlp_body(sc_ref, sb_ref, st_ref, f_ref, wpT_ref, bp_ref,
              w1T_ref, b1_ref, w2T_ref, b2_ref, out_ref):
    f32 = jnp.float32
    fe = jnp.dot(f_ref[...], wpT_ref[...], preferred_element_type=f32)
    fe = fe + bp_ref[...]
    xm = jnp.concatenate([sc_ref[:, :EMBED], sb_ref[:, :EMBED],
                          st_ref[:, :EMBED]], axis=1)
    x = jnp.concatenate([xm * (1.0 / SEQ), fe], axis=1)
    h = jnp.dot(x, w1T_ref[...], preferred_element_type=f32) + b1_ref[...]
    h = jnp.maximum(h, 0.0)
    out_ref[...] = jnp.dot(h, w2T_ref[...],
                           preferred_element_type=f32) + b2_ref[...]


def _mlp(Sc, Sb, St, f, WpT, bp2, W1T, b12, W2T, b22):
    nb = BATCH // BM
    full = lambda r, c: pl.BlockSpec((r, c), lambda i: (0, 0))
    return pl.pallas_call(
        _mlp_body,
        grid=(nb,),
        in_specs=[
            pl.BlockSpec((BM, PADE), lambda i: (i, 0)),
            pl.BlockSpec((BM, PADE), lambda i: (i, 0)),
            pl.BlockSpec((BM, PADE), lambda i: (i, 0)),
            pl.BlockSpec((BM, FEAT), lambda i: (i, 0)),
            full(FEAT, EMBED),
            full(1, EMBED),
            full(4 * EMBED, 128),
            full(1, 128),
            full(128, 2),
            full(1, 2),
        ],
        out_specs=pl.BlockSpec((BM, 2), lambda i: (i, 0)),
        out_shape=jax.ShapeDtypeStruct((BATCH, 2), jnp.float32),
    )(Sc, Sb, St, f, WpT, bp2, W1T, b12, W2T, b22)


def _pad_table(tab):
    return jnp.pad(tab, ((0, 0), (0, PADE - tab.shape[1])))


def kernel(c, b, t, f, char_emb, bi_emb, tri_emb, Wp, bp, W1, b1, W2, b2):
    c2 = c.astype(jnp.int32).reshape(-1, GW)
    b2i = b.astype(jnp.int32).reshape(-1, GW)
    t2 = t.astype(jnp.int32).reshape(-1, GW)
    Sc, Sb, St = _sc_pool_kernel(c2, b2i, t2, _pad_table(char_emb),
                                 _pad_table(bi_emb), _pad_table(tri_emb))
    return _mlp(Sc, Sb, St, f,
                Wp.T, bp.reshape(1, -1),
                W1.T, b1.reshape(1, -1),
                W2.T, b2.reshape(1, -1))


# speedup vs baseline: 9.8552x; 1.7567x over previous
import jax
import jax.numpy as jnp
from jax import lax
from jax.experimental import pallas as pl
from jax.experimental.pallas import tpu as pltpu
from jax.experimental.pallas import tpu_sc as plsc

EMBED = 64
PADE = 128
SEQ = 200
BATCH = 16384
FEAT = 32
NCHAR = 128

NC, NS = 2, 16
NW = NC * NS
RPT = BATCH // NW
IPT = RPT * SEQ
GW = 128
SPB = 8
BLK = SPB * GW
NBLK = IPT // BLK
IDX_ROWS_PT = IPT // GW
NGRP = RPT // 16
GSUB = 8
CSUB = GSUB * 16
ZR = 64
RHALF = RPT // 2


def _sc_pool_kernel(cT3, b2, t2, tab_b, tab_t):
    mesh = plsc.VectorSubcoreMesh(core_axis_name="c", subcore_axis_name="s",
                                  num_cores=NC, num_subcores=NS)

    def body(cT3_hbm, b_hbm, t_hbm, tabb, tabt,
             out_cnt, out_b, out_t,
             idx_v, cidx_v, rows0, rows1, zbuf, cnt, seg_v, seg0_v, acc,
             sem0, sem1, ssem0, ssem1):
        cid = lax.axis_index("c")
        sid = lax.axis_index("s")
        gtile = cid * NS + sid
        accbase = sid * RPT
        accbase2 = sid * RHALF
        gbase = gtile * RPT
        idx_row0 = gtile * IDX_ROWS_PT

        zero = jnp.zeros((16,), jnp.float32)

        def zbody(i, carry):
            for h in range(PADE // 16):
                zbuf[i, pl.ds(h * 16, 16)] = zero
            return carry

        lax.fori_loop(0, ZR, zbody, 0)
        for h in range(GW // 16):
            seg_v[pl.ds(h * 16, 16)] = accbase2 + lax.iota(jnp.int32, 16) * 0
        for h in range(ZR // 16):
            seg0_v[pl.ds(h * 16, 16)] = accbase2 + lax.iota(jnp.int32, 16) * 0

        ones = jnp.ones((16,), jnp.float32)
        lane_off = lax.iota(jnp.int32, 16) * NCHAR

        def sub_body(sub, carry):
            def czero(i, carry2):
                cnt[pl.ds(i * 16, 16)] = zero
                return carry2

            lax.fori_loop(0, CSUB * NCHAR // 16, czero, 0)

            def grp_body(g, carry2):
                pltpu.sync_copy(
                    cT3_hbm.at[pl.ds(
                        (gtile * NGRP + sub * GSUB + g) * SEQ, SEQ)], cidx_v)
                goff = g * (16 * NCHAR) + lane_off

                def s_body(s, carry3):
                    addr = goff + cidx_v[s, :]
                    prev = plsc.load_gather(cnt, [addr])
                    plsc.store_scatter(cnt, [addr], prev + ones)
                    return carry3

                lax.fori_loop(0, SEQ, s_body, 0)
                return carry2

            lax.fori_loop(0, GSUB, grp_body, 0)
            pltpu.sync_copy(
                cnt,
                out_cnt.at[pl.ds((gbase + sub * CSUB) * NCHAR, CSUB * NCHAR)])
            return carry

        lax.fori_loop(0, NGRP // GSUB, sub_body, 0)

        bufs = (rows0, rows1)
        gsems = (sem0, sem1)
        ssems = (ssem0, ssem1)

        def scat_wait(par):
            pltpu.make_async_copy(bufs[par], acc.at[seg_v], ssems[par]).wait()

        for idx_hbm, tab, out in (
            (b_hbm, tabb, out_b),
            (t_hbm, tabt, out_t),
        ):
            for half in range(2):
                for k in range(RHALF // ZR):
                    pltpu.sync_copy(zbuf,
                                    acc.at[pl.ds(accbase2 + k * ZR, ZR)])
                for _ in range(GW // ZR):
                    pltpu.async_copy(zbuf, acc.at[seg0_v], ssems[0],
                                     add=True)
                    pltpu.async_copy(zbuf, acc.at[seg0_v], ssems[1],
                                     add=True)
                idx_row_h = idx_row0 + half * (IDX_ROWS_PT // 2)

                def blk_body(blk, carry, idx_hbm=idx_hbm, tab=tab,
                             idx_row_h=idx_row_h):
                    pltpu.sync_copy(
                        idx_hbm.at[pl.ds(idx_row_h + blk * SPB, SPB)], idx_v)
                    descs = [None] * SPB
                    scat_wait(0)
                    descs[0] = pltpu.async_copy(tab.at[idx_v.at[0]], rows0,
                                                sem0)
                    for j in range(SPB):
                        if j + 1 < SPB:
                            scat_wait((j + 1) % 2)
                            descs[j + 1] = pltpu.async_copy(
                                tab.at[idx_v.at[j + 1]], bufs[(j + 1) % 2],
                                gsems[(j + 1) % 2])
                        descs[j].wait()
                        p0 = blk * BLK + j * GW
                        for h in range(GW // 16):
                            v = p0 + h * 16 + lax.iota(jnp.int32, 16)
                            q = lax.div(v, jnp.full((16,), SEQ, jnp.int32))
                            seg_v[pl.ds(h * 16, 16)] = accbase2 + q
                        pltpu.async_copy(bufs[j % 2], acc.at[seg_v],
                                         ssems[j % 2], add=True)
                    return carry

                lax.fori_loop(0, NBLK // 2, blk_body, 0)
                scat_wait(0)
                scat_wait(1)
                pltpu.sync_copy(
                    acc.at[pl.ds(accbase2, RHALF)],
                    out.at[pl.ds(gbase + half * RHALF, RHALF)])

    f32 = jnp.float32
    return pl.kernel(
        body,
        out_type=[jax.ShapeDtypeStruct((BATCH * NCHAR,), f32),
                  jax.ShapeDtypeStruct((BATCH, PADE), f32),
                  jax.ShapeDtypeStruct((BATCH, PADE), f32)],
        mesh=mesh,
        compiler_params=pltpu.CompilerParams(needs_layout_passes=False),
        scratch_types=[
            pltpu.VMEM((SPB, GW), jnp.int32),
            pltpu.VMEM((SEQ, 16), jnp.int32),
            pltpu.VMEM((GW, PADE), f32),
            pltpu.VMEM((GW, PADE), f32),
            pltpu.VMEM((ZR, PADE), f32),
            pltpu.VMEM((CSUB * NCHAR,), f32),
            pltpu.VMEM((GW,), jnp.int32),
            pltpu.VMEM((ZR,), jnp.int32),
            pltpu.VMEM_SHARED((NS * RHALF, PADE), f32),
            pltpu.SemaphoreType.DMA,
            pltpu.SemaphoreType.DMA,
            pltpu.SemaphoreType.DMA,
            pltpu.SemaphoreType.DMA,
        ],
    )(cT3, b2, t2, tab_b, tab_t)


BM = 512


def _mlp_body(cnt_ref, sb_ref, st_ref, f_ref, tabc_ref, wpT_ref, bp_ref,
              w1T_ref, b1_ref, w2T_ref, b2_ref, out_ref):
    f32 = jnp.float32
    ce = jnp.dot(cnt_ref[...], tabc_ref[...], preferred_element_type=f32)
    fe = jnp.dot(f_ref[...], wpT_ref[...], preferred_element_type=f32)
    fe = fe + bp_ref[...]
    xm = jnp.concatenate([ce, sb_ref[:, :EMBED], st_ref[:, :EMBED]], axis=1)
    x = jnp.concatenate([xm * (1.0 / SEQ), fe], axis=1)
    h = jnp.dot(x, w1T_ref[...], preferred_element_type=f32) + b1_ref[...]
    h = jnp.maximum(h, 0.0)
    out_ref[...] = jnp.dot(h, w2T_ref[...],
                           preferred_element_type=f32) + b2_ref[...]


def _mlp(cnt, Sb, St, f, tabc, WpT, bp2, W1T, b12, W2T, b22):
    nb = BATCH // BM
    full = lambda r, c: pl.BlockSpec((r, c), lambda i: (0, 0))
    return pl.pallas_call(
        _mlp_body,
        grid=(nb,),
        in_specs=[
            pl.BlockSpec((BM, NCHAR), lambda i: (i, 0)),
            pl.BlockSpec((BM, PADE), lambda i: (i, 0)),
            pl.BlockSpec((BM, PADE), lambda i: (i, 0)),
            pl.BlockSpec((BM, FEAT), lambda i: (i, 0)),
            full(NCHAR, EMBED),
            full(FEAT, EMBED),
            full(1, EMBED),
            full(4 * EMBED, 128),
            full(1, 128),
            full(128, 2),
            full(1, 2),
        ],
        out_specs=pl.BlockSpec((BM, 2), lambda i: (i, 0)),
        out_shape=jax.ShapeDtypeStruct((BATCH, 2), jnp.float32),
    )(cnt, Sb, St, f, tabc, WpT, bp2, W1T, b12, W2T, b22)


def _pad_table(tab):
    return jnp.pad(tab, ((0, 0), (0, PADE - tab.shape[1])))


def kernel(c, b, t, f, char_emb, bi_emb, tri_emb, Wp, bp, W1, b1, W2, b2):
    cT3 = (c.astype(jnp.int32)
           .reshape(NW, NGRP, 16, SEQ)
           .transpose(0, 1, 3, 2)
           .reshape(NW * NGRP * SEQ, 16))
    b2i = b.astype(jnp.int32).reshape(-1, GW)
    t2 = t.astype(jnp.int32).reshape(-1, GW)
    cnt, Sb, St = _sc_pool_kernel(cT3, b2i, t2, _pad_table(bi_emb),
                                  _pad_table(tri_emb))
    cnt = cnt.reshape(BATCH, NCHAR)
    tabc = jnp.pad(char_emb, ((0, NCHAR - char_emb.shape[0]), (0, 0)))
    return _mlp(cnt, Sb, St, f, tabc,
                Wp.T, bp.reshape(1, -1),
                W1.T, b1.reshape(1, -1),
                W2.T, b2.reshape(1, -1))
